# baseline (speedup 1.0000x reference)
import functools

import jax
import jax.numpy as jnp
from jax import lax
from jax.experimental import pallas as pl
from jax.experimental.pallas import tpu as pltpu
from jax.experimental.pallas import tpu_sc as plsc

_NUM_CORES = 2
_NUM_SUBCORES = 16
_NUM_WORKERS = _NUM_CORES * _NUM_SUBCORES
_LANES = 16
_BW = 128
_PACK = 4


def _sc_gather_t(tok_t, w4):
    hist, batch = tok_t.shape
    d = w4.shape[1] // _PACK
    span = batch // _NUM_WORKERS
    nchunk = span // _BW
    units = hist * nchunk
    half_h = hist // 2
    half_units = units // 2
    tstride = _BW + 1
    mesh = plsc.VectorSubcoreMesh(core_axis_name="c", subcore_axis_name="s")

    @functools.partial(
        pl.kernel,
        mesh=mesh,
        out_type=jax.ShapeDtypeStruct((hist, d, batch), jnp.float32),
        scratch_types=[
            pltpu.VMEM((half_h, span), jnp.int32),
            pltpu.VMEM((units * _BW,), jnp.int32),
            pltpu.VMEM((units * _BW,), jnp.int32),
            pltpu.VMEM((2, _BW, _PACK * d), jnp.float32),
            pltpu.VMEM((2, d, tstride), jnp.float32),
            pltpu.SemaphoreType.DMA((2,)),
            pltpu.SemaphoreType.DMA((2,)),
        ],
        compiler_params=pltpu.CompilerParams(
            use_tc_tiling_on_sc=False, needs_layout_passes=False
        ),
    )
    def k(tok_hbm, w4_hbm, out_hbm, idx_v, vrow_v, sel_v, rows_v, trans_v,
          sem_g, sem_s):
        wid = lax.axis_index("s") * _NUM_CORES + lax.axis_index("c")
        b0 = wid * span
        iota = jnp.arange(_LANES, dtype=jnp.int32)

        def stage_half(half):
            pltpu.sync_copy(
                tok_hbm.at[pl.ds(half * half_h, half_h), pl.ds(b0, span)], idx_v
            )
            base = half * half_units * _BW
            nvec = half_h * span // _LANES

            def pre(kk, carry):
                r = kk // (span // _LANES)
                c0 = (kk % (span // _LANES)) * _LANES
                rvec = jnp.full((_LANES,), r, dtype=jnp.int32)
                raw = plsc.load_gather(idx_v, [rvec, c0 + iota])
                pos = base + r * span + c0 + iota
                plsc.store_scatter(vrow_v, [pos], raw >> 2)
                plsc.store_scatter(sel_v, [pos], (raw & 3) * d)
                return carry

            plsc.parallel_loop(0, nvec, unroll=8)(lambda kk: pre(kk, None))

        stage_half(0)
        stage_half(1)

        def start_gather(u, b):
            off = pl.multiple_of(u * _BW, _BW)
            pltpu.async_copy(
                w4_hbm.at[vrow_v.at[pl.ds(off, _BW)]], rows_v.at[b], sem_g.at[b]
            )

        def wait_gather(u, b):
            off = pl.multiple_of(u * _BW, _BW)
            pltpu.make_async_copy(
                w4_hbm.at[vrow_v.at[pl.ds(off, _BW)]], rows_v.at[b], sem_g.at[b]
            ).wait()

        def transpose(u, b):
            rows = rows_v.at[b]
            trans = trans_v.at[b]
            ubase = u * _BW

            def one_block(j0, carry):
                off = pl.multiple_of(ubase + j0, _LANES)
                sel16 = sel_v[pl.ds(off, _LANES)]
                for jj in range(_LANES):
                    j = j0 + jj
                    jvec = jnp.full((_LANES,), j, dtype=jnp.int32)
                    svec = jnp.full((_LANES,), sel16[jj], dtype=jnp.int32)
                    for e0 in range(d // _LANES):
                        cols = e0 * _LANES + iota
                        vals = plsc.load_gather(rows, [jvec, cols + svec])
                        plsc.store_scatter(trans, [cols, jvec], vals)
                return carry

            plsc.parallel_loop(0, _BW, step=_LANES)(lambda j0: one_block(j0, None))

        def fire_stores(u, b):
            h = u // nchunk
            boff = b0 + (u % nchunk) * _BW
            pltpu.async_copy(
                trans_v.at[b, :, pl.ds(0, _BW)],
                out_hbm.at[h, :, pl.ds(boff, _BW)],
                sem_s.at[b],
            )

        def wait_stores(u, b):
            h = u // nchunk
            boff = b0 + (u % nchunk) * _BW
            pltpu.make_async_copy(
                trans_v.at[b, :, pl.ds(0, _BW)],
                out_hbm.at[h, :, pl.ds(boff, _BW)],
                sem_s.at[b],
            ).wait()

        start_gather(0, 0)
        wait_gather(0, 0)
        start_gather(1, 1)
        transpose(0, 0)
        fire_stores(0, 0)
        wait_gather(1, 1)
        start_gather(2, 0)
        transpose(1, 1)
        fire_stores(1, 1)

        def body(u, b):
            wait_gather(u, b)
            start_gather(u + 1, 1 - b)
            wait_stores(u - 2, b)
            transpose(u, b)
            fire_stores(u, b)

        def pair(g, carry):
            body(2 * g, 0)
            body(2 * g + 1, 1)
            return carry

        pl.loop(1, (units - 2) // 2)(lambda g: pair(g, None))

        body(units - 2, (units - 2) % 2)

        ul = units - 1
        bl = ul % 2
        wait_gather(ul, bl)
        wait_stores(ul - 2, bl)
        transpose(ul, bl)
        fire_stores(ul, bl)
        wait_stores(ul - 1, 1 - bl)
        wait_stores(ul, bl)

    return k(tok_t, w4)


def kernel(token_ids, weight):
    tok_t = token_ids.T.astype(jnp.int32)
    w4 = weight.reshape(weight.shape[0] // _PACK, _PACK * weight.shape[1])
    out_t = _sc_gather_t(tok_t, w4)
    return out_t.transpose(2, 0, 1)


# speedup vs baseline: 1.3121x; 1.3121x over previous
import functools

import jax
import jax.numpy as jnp
from jax import lax
from jax.experimental import pallas as pl
from jax.experimental.pallas import tpu as pltpu
from jax.experimental.pallas import tpu_sc as plsc

_NUM_CORES = 2
_NUM_SUBCORES = 16
_NUM_WORKERS = _NUM_CORES * _NUM_SUBCORES
_LANES = 16


def _sc_gather_t(tok3, weight):
    hist, nw, bw = tok3.shape
    batch = nw * bw
    d = weight.shape[1]
    tstride = bw + 1
    mesh = plsc.VectorSubcoreMesh(core_axis_name="c", subcore_axis_name="s")

    @functools.partial(
        pl.kernel,
        mesh=mesh,
        out_type=jax.ShapeDtypeStruct((hist, d, batch), jnp.float32),
        scratch_types=[
            pltpu.VMEM((hist, bw), jnp.int32),
            pltpu.VMEM((2, bw, d), jnp.float32),
            pltpu.VMEM((2, d, tstride), jnp.float32),
            pltpu.SemaphoreType.DMA((2,)),
            pltpu.SemaphoreType.DMA((2,)),
        ],
        compiler_params=pltpu.CompilerParams(
            use_tc_tiling_on_sc=False, needs_layout_passes=False
        ),
    )
    def k(tok_hbm, table_hbm, out_hbm, idx_v, rows_v, trans_v, sem_g, sem_s):
        wid = lax.axis_index("s") * _NUM_CORES + lax.axis_index("c")
        b0 = wid * bw
        pltpu.sync_copy(tok_hbm.at[:, wid], idx_v)

        def start_gather(h, b):
            pltpu.async_copy(table_hbm.at[idx_v.at[h]], rows_v.at[b], sem_g.at[b])

        def wait_gather(h, b):
            pltpu.make_async_copy(
                table_hbm.at[idx_v.at[h]], rows_v.at[b], sem_g.at[b]
            ).wait()

        iota = jnp.arange(_LANES, dtype=jnp.int32)

        def transpose(b):
            rows = rows_v.at[b]
            trans = trans_v.at[b]

            def one_row(j, carry):
                jvec = jnp.full((_LANES,), j, dtype=jnp.int32)
                for e0 in range(d // _LANES):
                    cols = e0 * _LANES + iota
                    vals = plsc.load_gather(rows, [jvec, cols])
                    plsc.store_scatter(trans, [cols, jvec], vals)
                return carry

            plsc.parallel_loop(0, bw, unroll=8)(lambda j: one_row(j, None))

        def fire_stores(h, b):
            pltpu.async_copy(
                trans_v.at[b, :, pl.ds(0, bw)],
                out_hbm.at[h, :, pl.ds(b0, bw)],
                sem_s.at[b],
            )

        def wait_stores(h, b):
            pltpu.make_async_copy(
                trans_v.at[b, :, pl.ds(0, bw)],
                out_hbm.at[h, :, pl.ds(b0, bw)],
                sem_s.at[b],
            ).wait()

        start_gather(0, 0)
        wait_gather(0, 0)
        start_gather(1, 1)
        transpose(0)
        fire_stores(0, 0)
        wait_gather(1, 1)
        start_gather(2, 0)
        transpose(1)
        fire_stores(1, 1)

        def body(h, b):
            wait_gather(h, b)
            start_gather(h + 1, 1 - b)
            wait_stores(h - 2, b)
            transpose(b)
            fire_stores(h, b)

        def pair(g, carry):
            body(2 * g, 0)
            body(2 * g + 1, 1)
            return carry

        pl.loop(1, (hist - 2) // 2)(lambda g: pair(g, None))

        body(hist - 2, (hist - 2) % 2)

        hl = hist - 1
        bl = hl % 2
        wait_gather(hl, bl)
        wait_stores(hl - 2, bl)
        transpose(bl)
        fire_stores(hl, bl)
        wait_stores(hl - 1, 1 - bl)
        wait_stores(hl, bl)

    return k(tok3, weight)


def kernel(token_ids, weight):
    hist = token_ids.shape[1]
    tok3 = token_ids.T.reshape(hist, _NUM_WORKERS, -1).astype(jnp.int32)
    out_t = _sc_gather_t(tok3, weight)
    return out_t.transpose(2, 0, 1)
